# baseline (speedup 1.0000x reference)
import jax
import jax.numpy as jnp
from jax.experimental import pallas as pl

_N = 10000
_R, _C = 8, 1250

_K0 = 0
_K1 = 42
_K2 = _K0 ^ _K1 ^ 0x1BD11BDA
_KS = (_K0, _K1, _K2)
_ROTS = ((13, 15, 26, 6), (17, 29, 16, 24))
_DROP_THRESH = 419430


def _drop_kernel(train_ref, test_ref, train_out, test_out):
    r = jax.lax.broadcasted_iota(jnp.uint32, (_R, _C), 0)
    c = jax.lax.broadcasted_iota(jnp.uint32, (_R, _C), 1)
    p = r * jnp.uint32(_C) + c
    x0 = jnp.full((_R, _C), jnp.uint32(_K0))
    x1 = p + jnp.uint32(_K1)
    for i in range(5):
        for d in _ROTS[i % 2]:
            x0 = x0 + x1
            x1 = (x1 << jnp.uint32(d)) | (x1 >> jnp.uint32(32 - d))
            x1 = x1 ^ x0
        x0 = x0 + jnp.uint32(_KS[(i + 1) % 3])
        x1 = x1 + jnp.uint32(_KS[(i + 2) % 3] + i + 1)
    bits = x0 ^ x1
    keep = (bits >> jnp.uint32(9)) > jnp.uint32(_DROP_THRESH)
    train_out[...] = jnp.logical_and(train_ref[...], keep)
    test_out[...] = jnp.logical_and(test_ref[...], keep)


def kernel(x, y, train_mask, test_mask, edge_index):
    tm = train_mask.reshape(_R, _C)
    sm = test_mask.reshape(_R, _C)
    to, so = pl.pallas_call(
        _drop_kernel,
        out_shape=(
            jax.ShapeDtypeStruct((_R, _C), jnp.bool_),
            jax.ShapeDtypeStruct((_R, _C), jnp.bool_),
        ),
    )(tm, sm)
    return (x, edge_index, y, to.reshape(_N), so.reshape(_N))
